# baseline (speedup 1.0000x reference)
import functools

import jax
import jax.numpy as jnp
from jax import lax
from jax.experimental import pallas as pl
from jax.experimental.pallas import tpu as pltpu
from jax.experimental.pallas import tpu_sc as plsc

R = 66
TEMPERATURE = 1.0
THRESHOLD = 0.05

NUM_WORKERS = 32
CHUNK = 128
D_PAD = 128


def _sc_gather(table, idx3d, chunks_per_worker):
    rows_per_worker = chunks_per_worker * CHUNK
    n_rows_out = NUM_WORKERS * rows_per_worker
    mesh = plsc.VectorSubcoreMesh(core_axis_name="c", subcore_axis_name="s")

    @functools.partial(
        pl.kernel,
        mesh=mesh,
        out_type=jax.ShapeDtypeStruct((n_rows_out, D_PAD), jnp.float32),
        scratch_types=[
            pltpu.VMEM((chunks_per_worker, CHUNK), jnp.int32),
            pltpu.VMEM((2, CHUNK, D_PAD), jnp.float32),
            pltpu.SemaphoreType.DMA,
            pltpu.SemaphoreType.DMA,
        ],
    )
    def gather_kernel(table_hbm, idx_hbm, out_hbm, idx_v, bufs, sem_g, sem_o):
        wid = lax.axis_index("s") * 2 + lax.axis_index("c")
        pltpu.sync_copy(idx_hbm.at[wid], idx_v)
        base = wid * rows_per_worker
        out_copies = []
        for j in range(chunks_per_worker):
            if j >= 2:
                out_copies[j - 2].wait()
            pltpu.async_copy(
                table_hbm.at[idx_v.at[j]], bufs.at[j % 2], sem_g).wait()
            out_copies.append(pltpu.async_copy(
                bufs.at[j % 2],
                out_hbm.at[pl.ds(base + j * CHUNK, CHUNK)],
                sem_o))
        for c in out_copies[-2:]:
            c.wait()

    return gather_kernel(table, idx3d)


def _loss_body(lg_ref, ga_ref, bh_ref, cf_ref, out_ref, *, n_rows):
    i = pl.program_id(0)
    a = jax.nn.log_sigmoid(lg_ref[...] / TEMPERATURE)
    g = jax.nn.log_sigmoid(ga_ref[...] / TEMPERATURE)

    m = bh_ref.shape[1]
    body_idx = bh_ref[0:1, :]
    head_idx = bh_ref[1:2, :]
    col_a = lax.broadcasted_iota(jnp.int32, (R, m), 0)
    col_g = lax.broadcasted_iota(jnp.int32, (D_PAD, m), 0)
    w1 = ((col_a == body_idx).astype(jnp.float32)
          - (col_a == head_idx).astype(jnp.float32))
    in_g = col_g >= 1
    w2 = ((in_g & (col_g + (R - 1) == body_idx)).astype(jnp.float32)
          - (in_g & (col_g + (R - 1) == head_idx)).astype(jnp.float32))

    diff = (jnp.dot(a.astype(jnp.bfloat16), w1.astype(jnp.bfloat16),
                    preferred_element_type=jnp.float32)
            + jnp.dot(g.astype(jnp.bfloat16), w2.astype(jnp.bfloat16),
                      preferred_element_type=jnp.float32))
    bias = jnp.log(cf_ref[0:1, :]) - THRESHOLD
    t = jnp.maximum(diff + bias, 0.0)

    @pl.when(i == 0)
    def _():
        out_ref[0, 0] = 0.0

    out_ref[0, 0] += jnp.sum(t) / n_rows


def _tc_loss(logits, gathered, bh_pad, cf_pad, blk):
    n_rows, _ = logits.shape
    m = bh_pad.shape[1]
    grid = n_rows // blk
    out = pl.pallas_call(
        functools.partial(_loss_body, n_rows=n_rows),
        grid=(grid,),
        in_specs=[
            pl.BlockSpec((blk, R), lambda i: (i, 0)),
            pl.BlockSpec((blk, D_PAD), lambda i: (i, 0)),
            pl.BlockSpec((8, m), lambda i: (0, 0)),
            pl.BlockSpec((8, m), lambda i: (0, 0)),
        ],
        out_specs=pl.BlockSpec(memory_space=pltpu.SMEM),
        out_shape=jax.ShapeDtypeStruct((1, 1), jnp.float32),
    )(logits, gathered, bh_pad, cf_pad)
    return out[0, 0]


def kernel(logits, anti_idx, body_head, confidence):
    n, r = logits.shape
    assert r == R
    m = body_head.shape[0]

    per_worker = -(-n // (NUM_WORKERS * CHUNK))
    n_pad_total = NUM_WORKERS * per_worker * CHUNK
    pad = jnp.arange(n_pad_total - n, dtype=jnp.int32)
    idx3d = jnp.concatenate([anti_idx.astype(jnp.int32), pad]).reshape(
        NUM_WORKERS, per_worker, CHUNK)
    table = jnp.pad(logits, ((0, 0), (0, D_PAD - R)))

    bh_pad = jnp.zeros((8, m), jnp.int32).at[:2, :].set(body_head.T)
    cf_pad = jnp.ones((8, m), jnp.float32).at[0:1, :].set(confidence.T)
    return _tc_loss(logits, table, bh_pad, cf_pad, blk=4464)
